# baseline (speedup 1.0000x reference)
import jax
import jax.numpy as jnp
from jax.experimental import pallas as pl
from jax.experimental.pallas import tpu as pltpu

_RES = 0.8
_POINT_Z = 0.8
_NB = 20
_NBATCH = 2


def _box_mask(bx, H, W, row0):
    c = jnp.cos(bx[:, 6])
    s = jnp.sin(bx[:, 6])
    k1 = c * bx[:, 0] + s * bx[:, 1]
    k2 = -s * bx[:, 0] + c * bx[:, 1]
    adx2 = jnp.abs(bx[:, 3]) * 0.5
    ady2 = jnp.abs(bx[:, 4]) * 0.5
    adz2 = jnp.abs(bx[:, 5]) * 0.5
    zok = jnp.abs(_POINT_Z - bx[:, 2]) <= adz2
    adx2 = jnp.where(zok, adx2, -1.0)

    xs_r = (jax.lax.broadcasted_iota(jnp.int32, (H, 1), 0).astype(jnp.float32)
            + (row0 - 128.0)) * _RES
    ys_c = (jax.lax.broadcasted_iota(jnp.int32, (1, W), 1).astype(jnp.float32)
            - W / 2.0) * _RES

    mask = None
    for nb in range(_NB):
        ax = c[nb] * xs_r - k1[nb]
        bxv = s[nb] * ys_c
        ay = -s[nb] * xs_r - k2[nb]
        byv = c[nb] * ys_c
        ins = (jnp.abs(ax + bxv) <= adx2[nb]) \
            & (jnp.abs(ay + byv) <= ady2[nb])
        mask = ins if mask is None else (mask | ins)
    return mask.astype(jnp.float32)


def _loss_kernel(boxes_ref, a_top, a_bot, o_top, o_bot, out_ref):
    Hh = a_top.shape[2]
    W = a_top.shape[3]

    for nb in range(_NBATCH):
        bx = boxes_ref[nb]
        inter = None
        union = None
        for a_ref, o_ref, row0 in (
                (a_top, o_top, 0.0), (a_bot, o_bot, float(Hh))):
            pred = jnp.sum(a_ref[nb], axis=0)
            orig = jnp.sum(o_ref[nb, 1:], axis=0)
            pred_g = (pred > 0.0).astype(jnp.float32)
            orig_g = (orig > 0.0).astype(jnp.float32)
            maskf = _box_mask(bx, Hh, W, row0)
            i_h = jnp.sum(pred_g * orig_g * maskf, keepdims=True)
            u_h = jnp.sum(jnp.maximum(pred_g, orig_g) * maskf, keepdims=True)
            inter = i_h if inter is None else inter + i_h
            union = u_h if union is None else union + u_h

        iou = inter / (union + 1e-6)
        out_ref[nb, :, :] = iou


def kernel(added_points, original_points, boxes, tf_ego):
    B, T, H, W = added_points.shape
    boxes_p = jnp.zeros((B, 32, 8), dtype=jnp.float32)
    boxes_p = boxes_p.at[:, : boxes.shape[1], :7].set(boxes)
    Hh = H // 2
    NB_ = _NBATCH

    out = pl.pallas_call(
        _loss_kernel,
        grid=(B // NB_,),
        in_specs=[
            pl.BlockSpec((NB_, 32, 8), lambda b: (b, 0, 0)),
            pl.BlockSpec((NB_, T, Hh, W), lambda b: (b, 0, 0, 0)),
            pl.BlockSpec((NB_, T, Hh, W), lambda b: (b, 0, 1, 0)),
            pl.BlockSpec((NB_, T + 1, Hh, W), lambda b: (b, 0, 0, 0)),
            pl.BlockSpec((NB_, T + 1, Hh, W), lambda b: (b, 0, 1, 0)),
        ],
        out_specs=pl.BlockSpec((NB_, 1, 1), lambda b: (b, 0, 0)),
        out_shape=jax.ShapeDtypeStruct((B, 1, 1), jnp.float32),
        compiler_params=pltpu.CompilerParams(
            dimension_semantics=("arbitrary",),
            vmem_limit_bytes=110 * 1024 * 1024,
        ),
    )(boxes_p, added_points, added_points,
      original_points, original_points)
    return jnp.sum(out) / B


# speedup vs baseline: 1.0655x; 1.0655x over previous
import jax
import jax.numpy as jnp
from jax.experimental import pallas as pl
from jax.experimental.pallas import tpu as pltpu

_RES = 0.8
_POINT_Z = 0.8
_NB = 20
_NSPLIT = 4


def _box_mask(bx, H, W, row0):
    c = jnp.cos(bx[:, 6])
    s = jnp.sin(bx[:, 6])
    k1 = c * bx[:, 0] + s * bx[:, 1]
    k2 = -s * bx[:, 0] + c * bx[:, 1]
    adx2 = jnp.abs(bx[:, 3]) * 0.5
    ady2 = jnp.abs(bx[:, 4]) * 0.5
    adz2 = jnp.abs(bx[:, 5]) * 0.5
    zok = jnp.abs(_POINT_Z - bx[:, 2]) <= adz2
    adx2 = jnp.where(zok, adx2, -1.0)

    xs_r = (jax.lax.broadcasted_iota(jnp.int32, (H, 1), 0).astype(jnp.float32)
            + (row0 - 128.0)) * _RES
    ys_c = (jax.lax.broadcasted_iota(jnp.int32, (1, W), 1).astype(jnp.float32)
            - W / 2.0) * _RES

    mask = None
    for nb in range(_NB):
        ax = c[nb] * xs_r - k1[nb]
        bxv = s[nb] * ys_c
        ay = -s[nb] * xs_r - k2[nb]
        byv = c[nb] * ys_c
        ins = (jnp.abs(ax + bxv) <= adx2[nb]) \
            & (jnp.abs(ay + byv) <= ady2[nb])
        mask = ins if mask is None else (mask | ins)
    return mask.astype(jnp.float32)


def _loss_kernel(boxes_ref, *refs):
    a_refs = refs[:_NSPLIT]
    o_refs = refs[_NSPLIT:2 * _NSPLIT]
    out_ref = refs[2 * _NSPLIT]
    Hs = a_refs[0].shape[2]
    W = a_refs[0].shape[3]

    bx = boxes_ref[0]
    inter = None
    union = None
    for i in range(_NSPLIT):
        pred = jnp.sum(a_refs[i][0], axis=0)
        orig = jnp.sum(o_refs[i][0, 1:], axis=0)
        pred_g = (pred > 0.0).astype(jnp.float32)
        orig_g = (orig > 0.0).astype(jnp.float32)
        maskf = _box_mask(bx, Hs, W, float(i * Hs))
        i_h = jnp.sum(pred_g * orig_g * maskf, keepdims=True)
        u_h = jnp.sum(jnp.maximum(pred_g, orig_g) * maskf, keepdims=True)
        inter = i_h if inter is None else inter + i_h
        union = u_h if union is None else union + u_h

    iou = inter / (union + 1e-6)
    out_ref[...] = iou[None]


def kernel(added_points, original_points, boxes, tf_ego):
    B, T, H, W = added_points.shape
    boxes_p = jnp.zeros((B, 32, 8), dtype=jnp.float32)
    boxes_p = boxes_p.at[:, : boxes.shape[1], :7].set(boxes)
    Hs = H // _NSPLIT

    def _a_spec(i):
        return pl.BlockSpec((1, T, Hs, W), lambda b, i=i: (b, 0, i, 0))

    def _o_spec(i):
        return pl.BlockSpec((1, T + 1, Hs, W), lambda b, i=i: (b, 0, i, 0))

    out = pl.pallas_call(
        _loss_kernel,
        grid=(B,),
        in_specs=[pl.BlockSpec((1, 32, 8), lambda b: (b, 0, 0))]
        + [_a_spec(i) for i in range(_NSPLIT)]
        + [_o_spec(i) for i in range(_NSPLIT)],
        out_specs=pl.BlockSpec((1, 1, 1), lambda b: (b, 0, 0)),
        out_shape=jax.ShapeDtypeStruct((B, 1, 1), jnp.float32),
        compiler_params=pltpu.CompilerParams(
            dimension_semantics=("arbitrary",),
            vmem_limit_bytes=110 * 1024 * 1024,
        ),
    )(boxes_p, *([added_points] * _NSPLIT), *([original_points] * _NSPLIT))
    return jnp.sum(out) / B


# speedup vs baseline: 1.1111x; 1.0428x over previous
import jax
import jax.numpy as jnp
from jax.experimental import pallas as pl
from jax.experimental.pallas import tpu as pltpu

_RES = 0.8
_POINT_Z = 0.8
_NB = 20
_NSPLIT = 8


def _box_mask(bx, H, W, row0):
    c = jnp.cos(bx[:, 6])
    s = jnp.sin(bx[:, 6])
    k1 = c * bx[:, 0] + s * bx[:, 1]
    k2 = -s * bx[:, 0] + c * bx[:, 1]
    adx2 = jnp.abs(bx[:, 3]) * 0.5
    ady2 = jnp.abs(bx[:, 4]) * 0.5
    adz2 = jnp.abs(bx[:, 5]) * 0.5
    zok = jnp.abs(_POINT_Z - bx[:, 2]) <= adz2
    adx2 = jnp.where(zok, adx2, -1.0)

    xs_r = (jax.lax.broadcasted_iota(jnp.int32, (H, 1), 0).astype(jnp.float32)
            + (row0 - 128.0)) * _RES
    ys_c = (jax.lax.broadcasted_iota(jnp.int32, (1, W), 1).astype(jnp.float32)
            - W / 2.0) * _RES

    mask = None
    for nb in range(_NB):
        ax = c[nb] * xs_r - k1[nb]
        bxv = s[nb] * ys_c
        ay = -s[nb] * xs_r - k2[nb]
        byv = c[nb] * ys_c
        ins = (jnp.abs(ax + bxv) <= adx2[nb]) \
            & (jnp.abs(ay + byv) <= ady2[nb])
        mask = ins if mask is None else (mask | ins)
    return mask.astype(jnp.float32)


def _loss_kernel(boxes_ref, *refs):
    a_refs = refs[:_NSPLIT]
    o_refs = refs[_NSPLIT:2 * _NSPLIT]
    out_ref = refs[2 * _NSPLIT]
    Hs = a_refs[0].shape[2]
    W = a_refs[0].shape[3]

    bx = boxes_ref[0]
    inter = None
    union = None
    for i in range(_NSPLIT):
        pred = jnp.sum(a_refs[i][0], axis=0)
        orig = jnp.sum(o_refs[i][0, 1:], axis=0)
        pred_g = (pred > 0.0).astype(jnp.float32)
        orig_g = (orig > 0.0).astype(jnp.float32)
        maskf = _box_mask(bx, Hs, W, float(i * Hs))
        i_h = jnp.sum(pred_g * orig_g * maskf, keepdims=True)
        u_h = jnp.sum(jnp.maximum(pred_g, orig_g) * maskf, keepdims=True)
        inter = i_h if inter is None else inter + i_h
        union = u_h if union is None else union + u_h

    iou = inter / (union + 1e-6)
    out_ref[...] = iou[None]


def kernel(added_points, original_points, boxes, tf_ego):
    B, T, H, W = added_points.shape
    boxes_p = jnp.zeros((B, 32, 8), dtype=jnp.float32)
    boxes_p = boxes_p.at[:, : boxes.shape[1], :7].set(boxes)
    Hs = H // _NSPLIT

    def _a_spec(i):
        return pl.BlockSpec((1, T, Hs, W), lambda b, i=i: (b, 0, i, 0))

    def _o_spec(i):
        return pl.BlockSpec((1, T + 1, Hs, W), lambda b, i=i: (b, 0, i, 0))

    out = pl.pallas_call(
        _loss_kernel,
        grid=(B,),
        in_specs=[pl.BlockSpec((1, 32, 8), lambda b: (b, 0, 0))]
        + [_a_spec(i) for i in range(_NSPLIT)]
        + [_o_spec(i) for i in range(_NSPLIT)],
        out_specs=pl.BlockSpec((1, 1, 1), lambda b: (b, 0, 0)),
        out_shape=jax.ShapeDtypeStruct((B, 1, 1), jnp.float32),
        compiler_params=pltpu.CompilerParams(
            dimension_semantics=("arbitrary",),
            vmem_limit_bytes=110 * 1024 * 1024,
        ),
    )(boxes_p, *([added_points] * _NSPLIT), *([original_points] * _NSPLIT))
    return jnp.sum(out) / B


# speedup vs baseline: 1.2900x; 1.1610x over previous
import jax
import jax.numpy as jnp
from jax.experimental import pallas as pl
from jax.experimental.pallas import tpu as pltpu

_RES = 0.8
_POINT_Z = 0.8
_NB = 20
_NSPLIT = 16


def _box_mask(bx, H, W, row0):
    c = jnp.cos(bx[:, 6])
    s = jnp.sin(bx[:, 6])
    k1 = c * bx[:, 0] + s * bx[:, 1]
    k2 = -s * bx[:, 0] + c * bx[:, 1]
    adx2 = jnp.abs(bx[:, 3]) * 0.5
    ady2 = jnp.abs(bx[:, 4]) * 0.5
    adz2 = jnp.abs(bx[:, 5]) * 0.5
    zok = jnp.abs(_POINT_Z - bx[:, 2]) <= adz2
    adx2 = jnp.where(zok, adx2, -1.0)

    xs_r = (jax.lax.broadcasted_iota(jnp.int32, (H, 1), 0).astype(jnp.float32)
            + (row0 - 128.0)) * _RES
    ys_c = (jax.lax.broadcasted_iota(jnp.int32, (1, W), 1).astype(jnp.float32)
            - W / 2.0) * _RES

    mask = None
    for nb in range(_NB):
        ax = c[nb] * xs_r - k1[nb]
        bxv = s[nb] * ys_c
        ay = -s[nb] * xs_r - k2[nb]
        byv = c[nb] * ys_c
        ins = (jnp.abs(ax + bxv) <= adx2[nb]) \
            & (jnp.abs(ay + byv) <= ady2[nb])
        mask = ins if mask is None else (mask | ins)
    return mask.astype(jnp.float32)


def _loss_kernel(boxes_ref, *refs):
    a_refs = refs[:_NSPLIT]
    o_refs = refs[_NSPLIT:2 * _NSPLIT]
    out_ref = refs[2 * _NSPLIT]
    Hs = a_refs[0].shape[2]
    W = a_refs[0].shape[3]

    bx = boxes_ref[0]
    inter = None
    union = None
    for i in range(_NSPLIT):
        pred = jnp.sum(a_refs[i][0], axis=0)
        orig = jnp.sum(o_refs[i][0, 1:], axis=0)
        pred_g = (pred > 0.0).astype(jnp.float32)
        orig_g = (orig > 0.0).astype(jnp.float32)
        maskf = _box_mask(bx, Hs, W, float(i * Hs))
        i_h = jnp.sum(pred_g * orig_g * maskf, keepdims=True)
        u_h = jnp.sum(jnp.maximum(pred_g, orig_g) * maskf, keepdims=True)
        inter = i_h if inter is None else inter + i_h
        union = u_h if union is None else union + u_h

    iou = inter / (union + 1e-6)
    out_ref[...] = iou[None]


def kernel(added_points, original_points, boxes, tf_ego):
    B, T, H, W = added_points.shape
    boxes_p = jnp.zeros((B, 32, 8), dtype=jnp.float32)
    boxes_p = boxes_p.at[:, : boxes.shape[1], :7].set(boxes)
    Hs = H // _NSPLIT

    def _a_spec(i):
        return pl.BlockSpec((1, T, Hs, W), lambda b, i=i: (b, 0, i, 0))

    def _o_spec(i):
        return pl.BlockSpec((1, T + 1, Hs, W), lambda b, i=i: (b, 0, i, 0))

    out = pl.pallas_call(
        _loss_kernel,
        grid=(B,),
        in_specs=[pl.BlockSpec((1, 32, 8), lambda b: (b, 0, 0))]
        + [_a_spec(i) for i in range(_NSPLIT)]
        + [_o_spec(i) for i in range(_NSPLIT)],
        out_specs=pl.BlockSpec((1, 1, 1), lambda b: (b, 0, 0)),
        out_shape=jax.ShapeDtypeStruct((B, 1, 1), jnp.float32),
        compiler_params=pltpu.CompilerParams(
            dimension_semantics=("arbitrary",),
            vmem_limit_bytes=110 * 1024 * 1024,
        ),
    )(boxes_p, *([added_points] * _NSPLIT), *([original_points] * _NSPLIT))
    return jnp.sum(out) / B
